# Initial kernel scaffold; baseline (speedup 1.0000x reference)
import jax
import jax.numpy as jnp
from jax.experimental import pallas as pl


def kernel(input, weights):
    raise NotImplementedError("write your pallas kernel here")


# speedup vs baseline: 1.9335x; 1.9335x over previous
import functools

import jax
import jax.numpy as jnp
from jax import lax
from jax.experimental import pallas as pl
from jax.experimental.pallas import tpu as pltpu
from jax.experimental.pallas import tpu_sc as plsc

_PAD = 1
_LANES = 16
_NW = 32
_K = 32


@functools.lru_cache(maxsize=None)
def _build_sc_kernel(B, S, D):
    TOKW = (B * S) // _NW
    WPR = S // TOKW
    NCHUNK = TOKW // _K
    mesh = plsc.VectorSubcoreMesh(core_axis_name="c", subcore_axis_name="s")

    @functools.partial(
        pl.kernel,
        out_type=jax.ShapeDtypeStruct((B * S, D), jnp.float32),
        mesh=mesh,
        scratch_types=[
            pltpu.VMEM((S,), jnp.int32),
            pltpu.VMEM((TOKW,), jnp.int32),
            pltpu.VMEM((_K, D), jnp.float32),
            pltpu.SemaphoreType.DMA,
        ],
        compiler_params=pltpu.CompilerParams(needs_layout_passes=False),
    )
    def sc_kernel(ids_hbm, table_hbm, out_hbm, ids_v, idx_v, rows_v, gsem):
        wid = lax.axis_index("s") * 2 + lax.axis_index("c")
        row = wid // WPR
        kk = wid % WPR

        pltpu.sync_copy(ids_hbm.at[pl.ds(row * S, S)], ids_v)

        one = jnp.full((_LANES,), 1, jnp.int32)
        zero = jnp.full((_LANES,), 0, jnp.int32)
        pad_vec = jnp.full((_LANES,), _PAD, jnp.int32)

        def _cnt(j, acc):
            v = ids_v[pl.ds(j * _LANES, _LANES)]
            mi = jnp.where(v != _PAD, one, zero)
            return acc + jnp.sum(mi)
        prefix = lax.fori_loop(0, kk * (TOKW // _LANES), _cnt, jnp.int32(0))

        chunk_off = kk * TOKW
        def _pos(j, run):
            v = ids_v[pl.ds(chunk_off + j * _LANES, _LANES)]
            m = v != _PAD
            mi = jnp.where(m, one, zero)
            c = jnp.cumsum(mi)
            idx_v[pl.ds(j * _LANES, _LANES)] = jnp.where(m, c + run, pad_vec)
            return run + jnp.sum(mi)
        lax.fori_loop(0, TOKW // _LANES, _pos, prefix + jnp.int32(1))

        out_base = wid * TOKW
        def _chunk(cix, carry):
            idxs = idx_v.at[pl.ds(cix * _K, _K)]
            pltpu.async_copy(table_hbm.at[idxs], rows_v, gsem).wait()
            pltpu.sync_copy(rows_v, out_hbm.at[pl.ds(out_base + cix * _K, _K)])
            return carry
        lax.fori_loop(0, NCHUNK, _chunk, jnp.int32(0))

    return sc_kernel


def kernel(input, weights):
    B, S = input.shape
    _, D = weights.shape
    out = _build_sc_kernel(B, S, D)(input.reshape(-1), weights)
    return out.reshape(B, S, D)
